# baseline (speedup 1.0000x reference)
import functools

import jax
import jax.numpy as jnp
from jax import lax
from jax.experimental import pallas as pl
from jax.experimental.pallas import tpu as pltpu
from jax.experimental.pallas import tpu_sc as plsc

N = 5000
NP = 5120
L = 16
NCHUNK = NP // L
CONF_T = 0.3
IOU_T = 0.25
IMG = 640.0


def _conf_body(scores_ref, cls_ref, out_ref):
    out_ref[...] = scores_ref[...] * jnp.max(cls_ref[...], axis=1, keepdims=True)


def _conf(scores2d, cls2d):
    return pl.pallas_call(
        _conf_body,
        out_shape=jax.ShapeDtypeStruct((NP, 1), jnp.float32),
    )(scores2d, cls2d)


_sc_mesh = plsc.VectorSubcoreMesh(
    core_axis_name="c", subcore_axis_name="s", num_cores=2, num_subcores=16)

_F32V = pltpu.VMEM((NP,), jnp.float32)


@functools.partial(
    pl.kernel,
    out_type=jax.ShapeDtypeStruct((NP,), jnp.float32),
    mesh=_sc_mesh,
    scratch_types=[_F32V] * 6,
    compiler_params=pltpu.CompilerParams(needs_layout_passes=False),
)
def _nms_sc(cx_h, cy_h, w_h, h_h, conf_h, keep_h,
            x1_v, y1_v, x2_v, y2_v, conf_v, keep_v):
    cid = lax.axis_index("c")
    sid = lax.axis_index("s")

    @pl.when((cid == 0) & (sid == 0))
    def _():
        pltpu.sync_copy(cx_h, x1_v)
        pltpu.sync_copy(cy_h, y1_v)
        pltpu.sync_copy(w_h, x2_v)
        pltpu.sync_copy(h_h, y2_v)
        pltpu.sync_copy(conf_h, conf_v)

        lane = jnp.arange(L, dtype=jnp.int32)
        zeros = jnp.zeros((L,), jnp.float32)

        def prep(k, _):
            s = pl.ds(k * L, L)
            cx = x1_v[s] * IMG
            cy = y1_v[s] * IMG
            w = x2_v[s] * IMG
            h = y2_v[s] * IMG
            x1_v[s] = cx - w / 2.0
            y1_v[s] = cy - h / 2.0
            x2_v[s] = cx + w / 2.0
            y2_v[s] = cy + h / 2.0
            keep_v[s] = zeros
            return 0
        lax.fori_loop(0, NCHUNK, prep, 0)

        def finish_argmax(m, ci):
            mx = jnp.max(m)
            cand = jnp.where(m == mx, ci * L + lane, jnp.int32(NP))
            return mx, jnp.min(cand)

        def init_pass(k, carry):
            m, ci = carry
            v = conf_v[pl.ds(k * L, L)]
            upd = v > m
            return jnp.where(upd, v, m), jnp.where(upd, k, ci)

        def body(state):
            _, gi = state
            giv = jnp.full((L,), gi, jnp.int32)
            plsc.store_scatter(keep_v, [giv], jnp.ones((L,), jnp.float32),
                               mask=lane == 0)
            px1 = plsc.load_gather(x1_v, [giv])
            py1 = plsc.load_gather(y1_v, [giv])
            px2 = plsc.load_gather(x2_v, [giv])
            py2 = plsc.load_gather(y2_v, [giv])
            parea = (px2 - px1) * (py2 - py1)

            def sweep(k, carry):
                m, ci = carry
                s = pl.ds(k * L, L)
                x1 = x1_v[s]
                y1 = y1_v[s]
                x2 = x2_v[s]
                y2 = y2_v[s]
                area = (x2 - x1) * (y2 - y1)
                iw = jnp.maximum(jnp.minimum(x2, px2) - jnp.maximum(x1, px1), 0.0)
                ih = jnp.maximum(jnp.minimum(y2, py2) - jnp.maximum(y1, py1), 0.0)
                inter = iw * ih
                iou = inter / (area + parea - inter + 1e-9)
                sup = (iou > IOU_T) | (k * L + lane == gi)
                v = jnp.where(sup, -1.0, conf_v[s])
                conf_v[s] = v
                upd = v > m
                return jnp.where(upd, v, m), jnp.where(upd, k, ci)

            m, ci = lax.fori_loop(
                0, NCHUNK, sweep,
                (jnp.full((L,), -3.0, jnp.float32), jnp.zeros((L,), jnp.int32)))
            return finish_argmax(m, ci)

        m0, ci0 = lax.fori_loop(
            0, NCHUNK, init_pass,
            (jnp.full((L,), -3.0, jnp.float32), jnp.zeros((L,), jnp.int32)))
        lax.while_loop(lambda st: st[0] > CONF_T, body, finish_argmax(m0, ci0))

        pltpu.sync_copy(keep_v, keep_h)


def _asm_body(boxes_ref, conf_ref, keep_ref, out_ref):
    xywh_norm = (boxes_ref[...] * IMG) / IMG
    k = keep_ref[...]
    out_ref[...] = jnp.concatenate([xywh_norm * k, conf_ref[...] * k], axis=1)


def _assemble(boxes, conf, keep):
    return pl.pallas_call(
        _asm_body,
        out_shape=jax.ShapeDtypeStruct((N, 5), jnp.float32),
    )(boxes, conf, keep)


def kernel(boxes, scores, cls_probs):
    scores2d = jnp.pad(scores, (0, NP - N)).reshape(NP, 1)
    cls_p = jnp.pad(cls_probs, ((0, NP - N), (0, 0)))
    conf_col = _conf(scores2d, cls_p)
    bp = jnp.pad(boxes, ((0, NP - N), (0, 0)))
    keep = _nms_sc(bp[:, 0], bp[:, 1], bp[:, 2], bp[:, 3],
                   conf_col.reshape(NP))
    out = _assemble(boxes, conf_col[:N], keep[:N].reshape(N, 1))
    return out


# speedup vs baseline: 5.2947x; 5.2947x over previous
import functools

import jax
import jax.numpy as jnp
from jax import lax
from jax.experimental import pallas as pl
from jax.experimental.pallas import tpu as pltpu
from jax.experimental.pallas import tpu_sc as plsc

N = 5000
NP = 5120
L = 16
NT = 16
SEG = NP // NT
SCHUNK = SEG // L
NCHUNK = NP // L
CAP = 512
CONF_T = 0.3
CONF_BITS = 0x3E99999A
IOU_T = 0.25
IMG = 640.0


def _conf_body(scores_ref, cls_ref, out_ref):
    out_ref[...] = scores_ref[...] * jnp.max(cls_ref[...], axis=1, keepdims=True)


def _conf(scores2d, cls2d):
    return pl.pallas_call(
        _conf_body,
        out_shape=jax.ShapeDtypeStruct((NP, 1), jnp.float32),
    )(scores2d, cls2d)


_sc_mesh = plsc.VectorSubcoreMesh(
    core_axis_name="c", subcore_axis_name="s", num_cores=2, num_subcores=16)


@functools.partial(
    pl.kernel,
    out_type=[jax.ShapeDtypeStruct((NP,), jnp.float32),
              jax.ShapeDtypeStruct((NP,), jnp.float32),
              jax.ShapeDtypeStruct((L,), jnp.int32)],
    mesh=_sc_mesh,
    scratch_types=[
        pltpu.VMEM((NP,), jnp.float32),
        pltpu.VMEM((NP,), jnp.float32),
        pltpu.VMEM((NP,), jnp.float32),
        pltpu.VMEM((NP,), jnp.float32),
        pltpu.VMEM((SEG,), jnp.float32),
        pltpu.VMEM((SEG,), jnp.float32),
        pltpu.VMEM((L,), jnp.int32),
        pltpu.VMEM((2, NT, L), jnp.int32),
        pltpu.VMEM_SHARED((2, NT, L), jnp.int32),
        pltpu.SMEM((1,), jnp.int32),
    ],
    compiler_params=pltpu.CompilerParams(needs_layout_passes=False),
)
def _nms_sc(cx_h, cy_h, w_h, h_h, conf_h, keep_h,
            conf_o, keep_o, stat_o,
            x1_v, y1_v, x2_v, y2_v, conf_v, keep_v, row_v, all_v, sh_x,
            cnt_s):
    cid = lax.axis_index("c")
    sid = lax.axis_index("s")

    @pl.when(cid == 0)
    def _():
        base = sid * SEG
        lane = jnp.arange(L, dtype=jnp.int32)

        pltpu.sync_copy(cx_h, x1_v)
        pltpu.sync_copy(cy_h, y1_v)
        pltpu.sync_copy(w_h, x2_v)
        pltpu.sync_copy(h_h, y2_v)
        pltpu.sync_copy(conf_h.at[pl.ds(base, SEG)], conf_v)
        pltpu.sync_copy(keep_h.at[pl.ds(base, SEG)], keep_v)

        @pl.when(sid == 0)
        def _():
            cnt_s[0] = jnp.int32(0)
        plsc.subcore_barrier()

        def prep(k, _):
            s = pl.ds(k * L, L)
            cx = x1_v[s] * IMG
            cy = y1_v[s] * IMG
            w = x2_v[s] * IMG
            h = y2_v[s] * IMG
            x1_v[s] = cx - w / 2.0
            y1_v[s] = cy - h / 2.0
            x2_v[s] = cx + w / 2.0
            y2_v[s] = cy + h / 2.0
            return 0
        lax.fori_loop(0, NCHUNK, prep, 0)

        def exchange(rnd, m, ci):
            keys = jnp.where(m >= 0.0, plsc.bitcast(m, jnp.int32),
                             jnp.int32(-1))
            lk = jnp.max(keys)
            gidx = base + ci * L + lane
            li = jnp.min(jnp.where(keys == lk, gidx, jnp.int32(NP)))
            slot = lax.rem(rnd, jnp.int32(2))
            row_v[...] = jnp.where(lane == 0, lk,
                                   jnp.where(lane == 1, li, jnp.int32(0)))
            pltpu.sync_copy(row_v, sh_x.at[slot, sid])
            plsc.fetch_and_add(cnt_s.at[0], jnp.int32(1), subcore_id=0)
            c = lax.while_loop(
                lambda c: c < rnd * NT,
                lambda c: plsc.fetch_and_add(cnt_s.at[0], jnp.int32(0),
                                             subcore_id=0),
                jnp.int32(0))
            j = lax.rem(c, jnp.int32(2))
            pltpu.sync_copy(sh_x.at[slot], all_v.at[j])
            jv = jnp.full((L,), j, jnp.int32)
            tv = plsc.load_gather(all_v, [jv, lane,
                                          jnp.zeros((L,), jnp.int32)])
            ti = plsc.load_gather(all_v, [jv, lane,
                                          jnp.ones((L,), jnp.int32)])
            gk = jnp.max(tv)
            gi = jnp.min(jnp.where(tv == gk, ti, jnp.int32(NP)))
            return gk, gi

        def init_pass():
            m = jnp.full((L,), -3.0, jnp.float32)
            ci = jnp.zeros((L,), jnp.int32)
            for k in range(SCHUNK):
                v = conf_v[pl.ds(k * L, L)]
                upd = v > m
                m = jnp.where(upd, v, m)
                ci = jnp.where(upd, k, ci)
            return exchange(jnp.int32(1), m, ci)

        def body(t, state):
            gk, gi = state
            live = gk > CONF_BITS
            gi = jnp.minimum(gi, jnp.int32(NP - 1))
            giv = jnp.full((L,), gi, jnp.int32)
            inseg = (gi >= base) & (gi < base + SEG) & live
            plsc.store_scatter(
                keep_v, [jnp.full((L,), jnp.clip(gi - base, 0, SEG - 1),
                                  jnp.int32)],
                jnp.ones((L,), jnp.float32), mask=(lane == 0) & inseg)
            px1 = plsc.load_gather(x1_v, [giv])
            py1 = plsc.load_gather(y1_v, [giv])
            px2 = plsc.load_gather(x2_v, [giv])
            py2 = plsc.load_gather(y2_v, [giv])
            parea = (px2 - px1) * (py2 - py1)

            m = jnp.full((L,), -3.0, jnp.float32)
            ci = jnp.zeros((L,), jnp.int32)
            for k in range(SCHUNK):
                s = pl.ds(k * L, L)
                g = pl.ds(base + k * L, L)
                x1 = x1_v[g]
                y1 = y1_v[g]
                x2 = x2_v[g]
                y2 = y2_v[g]
                area = (x2 - x1) * (y2 - y1)
                iw = jnp.maximum(jnp.minimum(x2, px2) - jnp.maximum(x1, px1),
                                 0.0)
                ih = jnp.maximum(jnp.minimum(y2, py2) - jnp.maximum(y1, py1),
                                 0.0)
                inter = iw * ih
                iou = inter / (area + parea - inter + 1e-9)
                sup = ((iou > IOU_T) | (base + k * L + lane == gi)) & live
                v = jnp.where(sup, -1.0, conf_v[s])
                conf_v[s] = v
                upd = v > m
                m = jnp.where(upd, v, m)
                ci = jnp.where(upd, k, ci)
            return exchange(t + jnp.int32(2), m, ci)

        gk, _ = lax.fori_loop(0, CAP, body, init_pass())

        pltpu.sync_copy(conf_v, conf_o.at[pl.ds(base, SEG)])
        pltpu.sync_copy(keep_v, keep_o.at[pl.ds(base, SEG)])

        @pl.when(sid == 0)
        def _():
            row_v[...] = jnp.full((L,), gk, jnp.int32)
            pltpu.sync_copy(row_v, stat_o)


def _asm_body(boxes_ref, conf_ref, keep_ref, out_ref):
    xywh_norm = (boxes_ref[...] * IMG) / IMG
    k = keep_ref[...]
    out_ref[...] = jnp.concatenate([xywh_norm * k, conf_ref[...] * k], axis=1)


def _assemble(boxes, conf, keep):
    return pl.pallas_call(
        _asm_body,
        out_shape=jax.ShapeDtypeStruct((N, 5), jnp.float32),
    )(boxes, conf, keep)


def kernel(boxes, scores, cls_probs):
    scores2d = jnp.pad(scores, (0, NP - N)).reshape(NP, 1)
    cls_p = jnp.pad(cls_probs, ((0, NP - N), (0, 0)))
    conf_col = _conf(scores2d, cls_p)
    bp = jnp.pad(boxes, ((0, NP - N), (0, 0)))
    cx, cy, w, h = bp[:, 0], bp[:, 1], bp[:, 2], bp[:, 3]

    st = _nms_sc(cx, cy, w, h, conf_col.reshape(NP),
                 jnp.zeros((NP,), jnp.float32))
    st = lax.while_loop(
        lambda c: c[2][0] > CONF_BITS,
        lambda c: _nms_sc(cx, cy, w, h, c[0], c[1]),
        st)
    keep = st[1]
    out = _assemble(boxes, conf_col[:N], keep[:N].reshape(N, 1))
    return out


# speedup vs baseline: 6.3498x; 1.1993x over previous
import functools

import jax
import jax.numpy as jnp
from jax import lax
from jax.experimental import pallas as pl
from jax.experimental.pallas import tpu as pltpu
from jax.experimental.pallas import tpu_sc as plsc

N = 5000
NP = 5120
L = 16
NT = 16
SEG = NP // NT
SCHUNK = SEG // L
NCHUNK = NP // L
CAP = 416
CONF_T = 0.3
CONF_BITS = 0x3E99999A
IOU_T = 0.25
IMG = 640.0


def _conf_body(scores_ref, cls_ref, out_ref):
    out_ref[...] = scores_ref[...] * jnp.max(cls_ref[...], axis=1, keepdims=True)


def _conf(scores2d, cls2d):
    return pl.pallas_call(
        _conf_body,
        out_shape=jax.ShapeDtypeStruct((NP, 1), jnp.float32),
    )(scores2d, cls2d)


_sc_mesh = plsc.VectorSubcoreMesh(
    core_axis_name="c", subcore_axis_name="s", num_cores=2, num_subcores=16)


@functools.partial(
    pl.kernel,
    out_type=[jax.ShapeDtypeStruct((NP,), jnp.float32),
              jax.ShapeDtypeStruct((NP,), jnp.float32),
              jax.ShapeDtypeStruct((L,), jnp.int32)],
    mesh=_sc_mesh,
    scratch_types=[
        pltpu.VMEM((NP,), jnp.float32),
        pltpu.VMEM((NP,), jnp.float32),
        pltpu.VMEM((NP,), jnp.float32),
        pltpu.VMEM((NP,), jnp.float32),
        pltpu.VMEM((SEG,), jnp.float32),
        pltpu.VMEM((SEG,), jnp.float32),
        pltpu.VMEM((L,), jnp.int32),
        pltpu.VMEM((2, NT, L), jnp.int32),
        pltpu.VMEM_SHARED((2, NT, L), jnp.int32),
        pltpu.SMEM((1,), jnp.int32),
    ],
    compiler_params=pltpu.CompilerParams(needs_layout_passes=False),
)
def _nms_sc(cx_h, cy_h, w_h, h_h, conf_h, keep_h,
            conf_o, keep_o, stat_o,
            x1_v, y1_v, x2_v, y2_v, conf_v, keep_v, row_v, all_v, sh_x,
            cnt_s):
    cid = lax.axis_index("c")
    sid = lax.axis_index("s")

    @pl.when(cid == 0)
    def _():
        base = sid * SEG
        lane = jnp.arange(L, dtype=jnp.int32)

        pltpu.sync_copy(cx_h, x1_v)
        pltpu.sync_copy(cy_h, y1_v)
        pltpu.sync_copy(w_h, x2_v)
        pltpu.sync_copy(h_h, y2_v)
        pltpu.sync_copy(conf_h.at[pl.ds(base, SEG)], conf_v)
        pltpu.sync_copy(keep_h.at[pl.ds(base, SEG)], keep_v)

        @pl.when(sid == 0)
        def _():
            cnt_s[0] = jnp.int32(0)
        plsc.subcore_barrier()

        def prep(k, _):
            s = pl.ds(k * L, L)
            cx = x1_v[s] * IMG
            cy = y1_v[s] * IMG
            w = x2_v[s] * IMG
            h = y2_v[s] * IMG
            x1_v[s] = cx - w / 2.0
            y1_v[s] = cy - h / 2.0
            x2_v[s] = cx + w / 2.0
            y2_v[s] = cy + h / 2.0
            return 0
        lax.fori_loop(0, NCHUNK, prep, 0)

        def exchange(rnd, m, ci):
            keys = jnp.where(m >= 0.0, plsc.bitcast(m, jnp.int32),
                             jnp.int32(-1))
            lk = jnp.max(keys)
            gidx = base + ci * L + lane
            li = jnp.min(jnp.where(keys == lk, gidx, jnp.int32(NP)))
            slot = lax.rem(rnd, jnp.int32(2))
            row_v[...] = jnp.where(lane == 0, lk,
                                   jnp.where(lane == 1, li, jnp.int32(0)))
            pltpu.sync_copy(row_v, sh_x.at[slot, sid])
            plsc.fetch_and_add(cnt_s.at[0], jnp.int32(1), subcore_id=0)
            c = lax.while_loop(
                lambda c: c < rnd * NT,
                lambda c: plsc.fetch_and_add(cnt_s.at[0], jnp.int32(0),
                                             subcore_id=0),
                jnp.int32(0))
            j = lax.rem(c, jnp.int32(2))
            pltpu.sync_copy(sh_x.at[slot], all_v.at[j])
            jv = jnp.full((L,), j, jnp.int32)
            tv = plsc.load_gather(all_v, [jv, lane,
                                          jnp.zeros((L,), jnp.int32)])
            ti = plsc.load_gather(all_v, [jv, lane,
                                          jnp.ones((L,), jnp.int32)])
            gk = jnp.max(tv)
            gi = jnp.min(jnp.where(tv == gk, ti, jnp.int32(NP)))
            return gk, gi

        def init_pass():
            m = jnp.full((L,), -3.0, jnp.float32)
            ci = jnp.zeros((L,), jnp.int32)
            for k in range(SCHUNK):
                v = conf_v[pl.ds(k * L, L)]
                upd = v > m
                m = jnp.where(upd, v, m)
                ci = jnp.where(upd, k, ci)
            return exchange(jnp.int32(1), m, ci)

        def body(t, state):
            gk, gi = state
            live = gk > CONF_BITS
            gi = jnp.minimum(gi, jnp.int32(NP - 1))
            giv = jnp.full((L,), gi, jnp.int32)
            inseg = (gi >= base) & (gi < base + SEG) & live
            plsc.store_scatter(
                keep_v, [jnp.full((L,), jnp.clip(gi - base, 0, SEG - 1),
                                  jnp.int32)],
                jnp.ones((L,), jnp.float32), mask=(lane == 0) & inseg)
            px1 = plsc.load_gather(x1_v, [giv])
            py1 = plsc.load_gather(y1_v, [giv])
            px2 = plsc.load_gather(x2_v, [giv])
            py2 = plsc.load_gather(y2_v, [giv])
            parea = (px2 - px1) * (py2 - py1)

            m = jnp.full((L,), -3.0, jnp.float32)
            ci = jnp.zeros((L,), jnp.int32)
            for k in range(SCHUNK):
                s = pl.ds(k * L, L)
                g = pl.ds(base + k * L, L)
                x1 = x1_v[g]
                y1 = y1_v[g]
                x2 = x2_v[g]
                y2 = y2_v[g]
                area = (x2 - x1) * (y2 - y1)
                iw = jnp.maximum(jnp.minimum(x2, px2) - jnp.maximum(x1, px1),
                                 0.0)
                ih = jnp.maximum(jnp.minimum(y2, py2) - jnp.maximum(y1, py1),
                                 0.0)
                inter = iw * ih
                iou = inter / (area + parea - inter + 1e-9)
                sup = ((iou > IOU_T) | (base + k * L + lane == gi)) & live
                v = jnp.where(sup, -1.0, conf_v[s])
                conf_v[s] = v
                upd = v > m
                m = jnp.where(upd, v, m)
                ci = jnp.where(upd, k, ci)
            return exchange(t + jnp.int32(2), m, ci)

        gk, _ = lax.fori_loop(0, CAP, body, init_pass())

        pltpu.sync_copy(conf_v, conf_o.at[pl.ds(base, SEG)])
        pltpu.sync_copy(keep_v, keep_o.at[pl.ds(base, SEG)])

        @pl.when(sid == 0)
        def _():
            row_v[...] = jnp.full((L,), gk, jnp.int32)
            pltpu.sync_copy(row_v, stat_o)


def _asm_body(boxes_ref, conf_ref, keep_ref, out_ref):
    xywh_norm = (boxes_ref[...] * IMG) / IMG
    k = keep_ref[...]
    out_ref[...] = jnp.concatenate([xywh_norm * k, conf_ref[...] * k], axis=1)


def _assemble(boxes, conf, keep):
    return pl.pallas_call(
        _asm_body,
        out_shape=jax.ShapeDtypeStruct((N, 5), jnp.float32),
    )(boxes, conf, keep)


def kernel(boxes, scores, cls_probs):
    scores2d = jnp.pad(scores, (0, NP - N)).reshape(NP, 1)
    cls_p = jnp.pad(cls_probs, ((0, NP - N), (0, 0)))
    conf_col = _conf(scores2d, cls_p)
    bp = jnp.pad(boxes, ((0, NP - N), (0, 0)))
    cx, cy, w, h = bp[:, 0], bp[:, 1], bp[:, 2], bp[:, 3]

    st = _nms_sc(cx, cy, w, h, conf_col.reshape(NP),
                 jnp.zeros((NP,), jnp.float32))
    st = lax.while_loop(
        lambda c: c[2][0] > CONF_BITS,
        lambda c: _nms_sc(cx, cy, w, h, c[0], c[1]),
        st)
    keep = st[1]
    out = _assemble(boxes, conf_col[:N], keep[:N].reshape(N, 1))
    return out


# speedup vs baseline: 7.0055x; 1.1033x over previous
import functools

import jax
import jax.numpy as jnp
from jax import lax
from jax.experimental import pallas as pl
from jax.experimental.pallas import tpu as pltpu
from jax.experimental.pallas import tpu_sc as plsc

N = 5000
NP = 5120
L = 16
NT = 16
SEG = NP // NT
SCHUNK = SEG // L
NCHUNK = NP // L
CAP = 416
CONF_T = 0.3
CONF_BITS = 0x3E99999A
IOU_T = 0.25
IMG = 640.0


def _conf_body(scores_ref, cls_ref, out_ref):
    out_ref[...] = scores_ref[...] * jnp.max(cls_ref[...], axis=1, keepdims=True)


def _conf(scores2d, cls2d):
    return pl.pallas_call(
        _conf_body,
        out_shape=jax.ShapeDtypeStruct((NP, 1), jnp.float32),
    )(scores2d, cls2d)


_sc_mesh = plsc.VectorSubcoreMesh(
    core_axis_name="c", subcore_axis_name="s", num_cores=2, num_subcores=16)


@functools.partial(
    pl.kernel,
    out_type=[jax.ShapeDtypeStruct((NP,), jnp.float32),
              jax.ShapeDtypeStruct((NP,), jnp.float32),
              jax.ShapeDtypeStruct((L,), jnp.int32)],
    mesh=_sc_mesh,
    scratch_types=[
        pltpu.VMEM((NP,), jnp.float32),
        pltpu.VMEM((NP,), jnp.float32),
        pltpu.VMEM((NP,), jnp.float32),
        pltpu.VMEM((NP,), jnp.float32),
        pltpu.VMEM((SEG,), jnp.float32),
        pltpu.VMEM((SEG,), jnp.float32),
        pltpu.VMEM((L,), jnp.int32),
        pltpu.VMEM((2, NT, L), jnp.int32),
        pltpu.VMEM_SHARED((2, NT, L), jnp.int32),
        pltpu.SMEM((1,), jnp.int32),
    ],
    compiler_params=pltpu.CompilerParams(needs_layout_passes=False),
)
def _nms_sc(cx_h, cy_h, w_h, h_h, conf_h, keep_h,
            conf_o, keep_o, stat_o,
            x1_v, y1_v, x2_v, y2_v, conf_v, keep_v, row_v, all_v, sh_x,
            cnt_s):
    cid = lax.axis_index("c")
    sid = lax.axis_index("s")

    @pl.when(cid == 0)
    def _():
        base = sid * SEG
        lane = jnp.arange(L, dtype=jnp.int32)

        pltpu.sync_copy(cx_h, x1_v)
        pltpu.sync_copy(cy_h, y1_v)
        pltpu.sync_copy(w_h, x2_v)
        pltpu.sync_copy(h_h, y2_v)
        pltpu.sync_copy(conf_h.at[pl.ds(base, SEG)], conf_v)
        pltpu.sync_copy(keep_h.at[pl.ds(base, SEG)], keep_v)

        @pl.when(sid == 0)
        def _():
            cnt_s[0] = jnp.int32(0)
        plsc.subcore_barrier()

        def prep(k, _):
            s = pl.ds(k * L, L)
            cx = x1_v[s] * IMG
            cy = y1_v[s] * IMG
            w = x2_v[s] * IMG
            h = y2_v[s] * IMG
            x1_v[s] = cx - w / 2.0
            y1_v[s] = cy - h / 2.0
            x2_v[s] = cx + w / 2.0
            y2_v[s] = cy + h / 2.0
            return 0
        lax.fori_loop(0, NCHUNK, prep, 0)

        def exchange(rnd, m, ci):
            keys = jnp.where(m >= 0.0, plsc.bitcast(m, jnp.int32),
                             jnp.int32(-1))
            lk = jnp.max(keys)
            gidx = base + ci * L + lane
            li = jnp.min(jnp.where(keys == lk, gidx, jnp.int32(NP)))
            slot = lax.rem(rnd, jnp.int32(2))
            row_v[...] = jnp.where(lane == 0, lk,
                                   jnp.where(lane == 1, li, jnp.int32(0)))
            pltpu.sync_copy(row_v, sh_x.at[slot, sid])
            plsc.fetch_and_add(cnt_s.at[0], jnp.int32(1), subcore_id=0)
            c = lax.while_loop(
                lambda c: c < rnd * NT,
                lambda c: plsc.fetch_and_add(cnt_s.at[0], jnp.int32(0),
                                             subcore_id=0),
                jnp.int32(0))
            j = lax.rem(c, jnp.int32(2))
            pltpu.sync_copy(sh_x.at[slot], all_v.at[j])
            jv = jnp.full((L,), j, jnp.int32)
            tv = plsc.load_gather(all_v, [jv, lane,
                                          jnp.zeros((L,), jnp.int32)])
            ti = plsc.load_gather(all_v, [jv, lane,
                                          jnp.ones((L,), jnp.int32)])
            gk = jnp.max(tv)
            gi = jnp.min(jnp.where(tv == gk, ti, jnp.int32(NP)))
            return gk, gi

        def init_pass():
            m = jnp.full((L,), -3.0, jnp.float32)
            ci = jnp.zeros((L,), jnp.int32)
            for k in range(SCHUNK):
                v = conf_v[pl.ds(k * L, L)]
                upd = v > m
                m = jnp.where(upd, v, m)
                ci = jnp.where(upd, k, ci)
            return exchange(jnp.int32(1), m, ci)

        def body(t, state):
            gk, gi = state
            live = gk > CONF_BITS
            gi = jnp.minimum(gi, jnp.int32(NP - 1))
            giv = jnp.full((L,), gi, jnp.int32)
            inseg = (gi >= base) & (gi < base + SEG) & live
            plsc.store_scatter(
                keep_v, [jnp.full((L,), jnp.clip(gi - base, 0, SEG - 1),
                                  jnp.int32)],
                jnp.ones((L,), jnp.float32), mask=(lane == 0) & inseg)
            px1 = plsc.load_gather(x1_v, [giv])
            py1 = plsc.load_gather(y1_v, [giv])
            px2 = plsc.load_gather(x2_v, [giv])
            py2 = plsc.load_gather(y2_v, [giv])
            parea = (px2 - px1) * (py2 - py1)

            m = jnp.full((L,), -3.0, jnp.float32)
            ci = jnp.zeros((L,), jnp.int32)
            for k in range(SCHUNK):
                s = pl.ds(k * L, L)
                g = pl.ds(base + k * L, L)
                x1 = x1_v[g]
                y1 = y1_v[g]
                x2 = x2_v[g]
                y2 = y2_v[g]
                area = (x2 - x1) * (y2 - y1)
                iw = jnp.maximum(jnp.minimum(x2, px2) - jnp.maximum(x1, px1),
                                 0.0)
                ih = jnp.maximum(jnp.minimum(y2, py2) - jnp.maximum(y1, py1),
                                 0.0)
                inter = iw * ih
                u = area + parea - inter + 1e-9
                d = inter * 67108864.0 - u * 16777216.0
                sup = ((d > u) | (base + k * L + lane == gi)) & live
                v = jnp.where(sup, -1.0, conf_v[s])
                conf_v[s] = v
                upd = v > m
                m = jnp.where(upd, v, m)
                ci = jnp.where(upd, k, ci)
            return exchange(t + jnp.int32(2), m, ci)

        gk, _ = lax.fori_loop(0, CAP, body, init_pass())

        pltpu.sync_copy(conf_v, conf_o.at[pl.ds(base, SEG)])
        pltpu.sync_copy(keep_v, keep_o.at[pl.ds(base, SEG)])

        @pl.when(sid == 0)
        def _():
            row_v[...] = jnp.full((L,), gk, jnp.int32)
            pltpu.sync_copy(row_v, stat_o)


def _asm_body(boxes_ref, conf_ref, keep_ref, out_ref):
    xywh_norm = (boxes_ref[...] * IMG) / IMG
    k = keep_ref[...]
    out_ref[...] = jnp.concatenate([xywh_norm * k, conf_ref[...] * k], axis=1)


def _assemble(boxes, conf, keep):
    return pl.pallas_call(
        _asm_body,
        out_shape=jax.ShapeDtypeStruct((N, 5), jnp.float32),
    )(boxes, conf, keep)


def kernel(boxes, scores, cls_probs):
    scores2d = jnp.pad(scores, (0, NP - N)).reshape(NP, 1)
    cls_p = jnp.pad(cls_probs, ((0, NP - N), (0, 0)))
    conf_col = _conf(scores2d, cls_p)
    bp = jnp.pad(boxes, ((0, NP - N), (0, 0)))
    cx, cy, w, h = bp[:, 0], bp[:, 1], bp[:, 2], bp[:, 3]

    st = _nms_sc(cx, cy, w, h, conf_col.reshape(NP),
                 jnp.zeros((NP,), jnp.float32))
    st = lax.while_loop(
        lambda c: c[2][0] > CONF_BITS,
        lambda c: _nms_sc(cx, cy, w, h, c[0], c[1]),
        st)
    keep = st[1]
    out = _assemble(boxes, conf_col[:N], keep[:N].reshape(N, 1))
    return out
